# baseline (speedup 1.0000x reference)
import functools

import jax
import jax.numpy as jnp
from jax.experimental import pallas as pl
from jax.experimental.pallas import tpu as pltpu

NUM_OUT_CHANNELS = 768


def _body(x_hbm, o_hbm, zbuf, csem, zsem, *, B, C):
    copies = []
    for b in range(B):
        cp = pltpu.make_async_copy(
            x_hbm.at[b], o_hbm.at[b, pl.ds(0, C)], csem.at[b]
        )
        cp.start()
        copies.append(cp)

    zbuf[...] = jnp.zeros_like(zbuf)
    zeros = []
    for b in range(B):
        zp = pltpu.make_async_copy(
            zbuf, o_hbm.at[b, pl.ds(C, NUM_OUT_CHANNELS - C)], zsem.at[b]
        )
        zp.start()
        zeros.append(zp)

    for cp in copies:
        cp.wait()
    for zp in zeros:
        zp.wait()


def kernel(x):
    B, C, H, W = x.shape
    body = functools.partial(_body, B=B, C=C)
    return pl.pallas_call(
        body,
        in_specs=[pl.BlockSpec(memory_space=pl.ANY)],
        out_specs=pl.BlockSpec(memory_space=pl.ANY),
        out_shape=jax.ShapeDtypeStruct((B, NUM_OUT_CHANNELS, H, W), x.dtype),
        scratch_shapes=[
            pltpu.VMEM((NUM_OUT_CHANNELS - C, H, W), x.dtype),
            pltpu.SemaphoreType.DMA((B,)),
            pltpu.SemaphoreType.DMA((B,)),
        ],
    )(x)


# speedup vs baseline: 10.5704x; 10.5704x over previous
import jax
import jax.numpy as jnp
from jax.experimental import pallas as pl

NUM_OUT_CHANNELS = 768
CB = 128


def _body(x_ref, o_ref, *, ncopy):
    c = pl.program_id(1)

    @pl.when(c < ncopy)
    def _copy():
        o_ref[...] = x_ref[...]

    @pl.when(c >= ncopy)
    def _zero():
        o_ref[...] = jnp.zeros_like(o_ref)


def kernel(x):
    B, C, H, W = x.shape
    ncopy = C // CB
    ntot = NUM_OUT_CHANNELS // CB

    import functools
    body = functools.partial(_body, ncopy=ncopy)

    return pl.pallas_call(
        body,
        grid=(B, ntot),
        in_specs=[
            pl.BlockSpec(
                (1, CB, H, W),
                lambda b, c: (b, jnp.minimum(c, ncopy - 1), 0, 0),
            )
        ],
        out_specs=pl.BlockSpec((1, CB, H, W), lambda b, c: (b, c, 0, 0)),
        out_shape=jax.ShapeDtypeStruct((B, NUM_OUT_CHANNELS, H, W), x.dtype),
    )(x)
